# baseline (speedup 1.0000x reference)
import jax
import jax.numpy as jnp
from jax import lax
from jax.experimental import pallas as pl
from jax.experimental.pallas import tpu as pltpu

U, S, F, D = 27, 1024, 26, 64
FD = F * D


def _body(m_ref, e_ref, bs_ref, ue_ref, fl_ref, out_ref):
    u = pl.program_id(0)
    u_ge1 = u >= 1

    ue_row = jnp.where(u_ge1, ue_ref[1:2, :], ue_ref[0:1, :])
    lane = lax.broadcasted_iota(jnp.int32, (1, FD), 1)
    fid = lax.shift_right_logical(lane, 6)
    mask = (fid == (u - 1)) & u_ge1
    fl_row = jnp.where(mask, fl_ref[1:2, :], fl_ref[0:1, :])
    base_row = bs_ref[...] + ue_row + fl_row

    acc = jnp.dot(m_ref[0], e_ref[...], preferred_element_type=jnp.float32)
    out_ref[0] = acc + base_row


@jax.jit
def kernel(m_data, W_val, b_val, feature_embed, universe_embed, intervention_flag):
    f32 = jnp.float32
    E = (jnp.eye(F, dtype=f32)[:, :, None] * W_val[0][None, None, :]).reshape(F, FD)
    base_static = feature_embed.reshape(1, FD) + jnp.tile(b_val, F)[None, :]
    ue_t = jnp.tile(universe_embed, (1, F))
    fl_t = jnp.tile(intervention_flag, (1, F))

    out = pl.pallas_call(
        _body,
        grid=(U,),
        in_specs=[
            pl.BlockSpec((1, S, F), lambda u: (u, 0, 0)),
            pl.BlockSpec((F, FD), lambda u: (0, 0)),
            pl.BlockSpec((1, FD), lambda u: (0, 0)),
            pl.BlockSpec((2, FD), lambda u: (0, 0)),
            pl.BlockSpec((2, FD), lambda u: (0, 0)),
        ],
        out_specs=pl.BlockSpec((1, S, FD), lambda u: (u, 0, 0)),
        out_shape=jax.ShapeDtypeStruct((U, S, FD), f32),
        compiler_params=pltpu.CompilerParams(
            dimension_semantics=("arbitrary",),
        ),
    )(m_data, E, base_static, ue_t, fl_t)
    return out.reshape(U, S * F, D)
